# baseline (speedup 1.0000x reference)
import functools

import jax
import jax.numpy as jnp
from jax import lax
from jax.experimental import pallas as pl
from jax.experimental.pallas import tpu as pltpu
from jax.experimental.pallas import tpu_sc as plsc

_NC = 2
_NS = 16
_NW = _NC * _NS


def _tc_body(pos_ref, k_ref, ok_ref, *, bs, P):
    base = pl.program_id(1) * bs
    ok_ref[...] = jnp.zeros_like(ok_ref)

    def step(p, c):
        t = pos_ref[p] - base

        @pl.when((t >= 0) & (t < bs))
        def _():
            ok_ref[0, pl.ds(t, 1), :] = k_ref[0, pl.ds(p, 1), :]

        return c

    jax.lax.fori_loop(0, P, step, 0)


def _tc_scatter(pos_adj, k2, S, *, bs=1024):
    B, P, HD = k2.shape
    return pl.pallas_call(
        functools.partial(_tc_body, bs=bs, P=P),
        grid_spec=pltpu.PrefetchScalarGridSpec(
            num_scalar_prefetch=1,
            grid=(B, S // bs),
            in_specs=[pl.BlockSpec((1, P, HD), lambda b, s, pref: (b, 0, 0))],
            out_specs=[pl.BlockSpec((1, bs, HD), lambda b, s, pref: (b, s, 0))],
        ),
        out_shape=[jax.ShapeDtypeStruct((B, S, HD), jnp.float32)],
        compiler_params=pltpu.CompilerParams(
            dimension_semantics=("parallel", "parallel"),
        ),
    )(pos_adj, k2)[0]


def _sc_scatter(pos_adj, p_eff, src2, B, S, P, HD):
    R = B * S
    rows_per_w = R // _NW
    q_per_b = S // rows_per_w
    zrows = 64
    mesh = plsc.VectorSubcoreMesh(core_axis_name="c", subcore_axis_name="s")

    @functools.partial(
        pl.kernel,
        mesh=mesh,
        out_type=jax.ShapeDtypeStruct((R, HD), jnp.float32),
        scratch_types=[
            pltpu.VMEM((zrows, HD), jnp.float32),
            pltpu.VMEM((P, HD), jnp.float32),
            pltpu.VMEM((P,), jnp.int32),
            pltpu.VMEM((P,), jnp.int32),
            pltpu.VMEM((P,), jnp.int32),
            pltpu.VMEM((P,), jnp.int32),
            pltpu.SemaphoreType.DMA,
        ],
        compiler_params=pltpu.CompilerParams(needs_layout_passes=False),
    )
    def sc_kernel(pa_hbm, pe_hbm, src_hbm, out_hbm,
                  zbuf, rows_v, pav, pev, didx, sidx, sem):
        c = lax.axis_index("c")
        s = lax.axis_index("s")
        w = s * _NC + c
        b = w // q_per_b
        q = w % q_per_b
        lo = q * rows_per_w
        row0 = b * S + lo

        zv = jnp.zeros((16,), jnp.float32)

        def zrow(i, carry):
            for j in range(HD // 16):
                zbuf[i, pl.ds(j * 16, 16)] = zv
            return carry

        lax.fori_loop(0, zrows, zrow, 0)

        zcopies = [
            pltpu.async_copy(
                zbuf, out_hbm.at[pl.ds(row0 + i * zrows, zrows)], sem)
            for i in range(rows_per_w // zrows)
        ]

        pltpu.sync_copy(pa_hbm, pav)
        pltpu.sync_copy(pe_hbm, pev)
        iota = lax.iota(jnp.int32, 16)
        pa0 = pav[pl.ds(0, 16)]
        pa1 = pav[pl.ds(16, 16)]
        pe0 = pev[pl.ds(0, 16)]
        pe1 = pev[pl.ds(16, 16)]
        m0 = (pa0 >= lo) & (pa0 < lo + rows_per_w)
        m1 = (pa1 >= lo) & (pa1 < lo + rows_per_w)
        any_mine = jnp.maximum(
            jnp.max(jnp.where(m0, 1, 0)), jnp.max(jnp.where(m1, 1, 0))) > 0
        lm = jnp.maximum(jnp.max(jnp.where(m0, iota, -1)),
                         jnp.max(jnp.where(m1, iota + 16, -1)))
        trash_s = jnp.maximum(jnp.max(jnp.where(iota == lm, pa0, -1)),
                              jnp.max(jnp.where(iota + 16 == lm, pa1, -1)))
        trash_src = jnp.maximum(jnp.max(jnp.where(iota == lm, pe0, -1)),
                                jnp.max(jnp.where(iota + 16 == lm, pe1, -1)))
        didx[pl.ds(0, 16)] = b * S + jnp.where(m0, pa0, trash_s)
        didx[pl.ds(16, 16)] = b * S + jnp.where(m1, pa1, trash_s)
        sidx[pl.ds(0, 16)] = b * P + jnp.where(m0, pe0, trash_src)
        sidx[pl.ds(16, 16)] = b * P + jnp.where(m1, pe1, trash_src)

        for zc in zcopies:
            zc.wait()

        @pl.when(any_mine)
        def _():
            pltpu.async_copy(src_hbm.at[sidx], rows_v, sem).wait()
            pltpu.async_copy(rows_v, out_hbm.at[didx], sem).wait()

    return sc_kernel(pos_adj, p_eff, src2)


def kernel(k, v, pos, max_pos, k_cache, v_cache):
    B, P, H, D = k.shape
    S = k_cache.shape[1]
    HD = H * D
    pos_i = pos.astype(jnp.int32) % S
    r = (jnp.asarray(max_pos, jnp.int32) + 1) % S
    pos_adj = (pos_i - r) % S
    nxt = jnp.concatenate([pos_adj[1:], jnp.full((1,), -1, jnp.int32)])
    idx = jnp.arange(P, dtype=jnp.int32)
    cand = jnp.where(pos_adj != nxt, idx, jnp.int32(P))
    p_eff = jnp.flip(lax.cummin(jnp.flip(cand)))

    ok = _tc_scatter(pos_adj, k.reshape(B, P, HD), S)
    ov = _sc_scatter(pos_adj, p_eff, v.reshape(B * P, HD), B, S, P, HD)
    return ok.reshape(B, S, H, D), ov.reshape(B, S, H, D)


# speedup vs baseline: 1.0865x; 1.0865x over previous
import functools

import jax
import jax.numpy as jnp
from jax.experimental import pallas as pl
from jax.experimental.pallas import tpu as pltpu


def _scatter_body(pos_ref, k_ref, v_ref, ok_ref, ov_ref, *, bs, P):
    base = pl.program_id(1) * bs
    ok_ref[...] = jnp.zeros_like(ok_ref)
    ov_ref[...] = jnp.zeros_like(ov_ref)

    def step(p, c):
        t = pos_ref[p] - base

        @pl.when((t >= 0) & (t < bs))
        def _():
            ok_ref[0, pl.ds(t, 1)] = k_ref[0, pl.ds(p, 1)]
            ov_ref[0, pl.ds(t, 1)] = v_ref[0, pl.ds(p, 1)]

        return c

    jax.lax.fori_loop(0, P, step, 0)


def _scatter_full(pos_adj, k, v, S, *, bs=1024):
    B, P, H, D = k.shape
    return pl.pallas_call(
        functools.partial(_scatter_body, bs=bs, P=P),
        grid_spec=pltpu.PrefetchScalarGridSpec(
            num_scalar_prefetch=1,
            grid=(B, S // bs),
            in_specs=[
                pl.BlockSpec((1, P, H, D), lambda b, s, pref: (b, 0, 0, 0)),
                pl.BlockSpec((1, P, H, D), lambda b, s, pref: (b, 0, 0, 0)),
            ],
            out_specs=[
                pl.BlockSpec((1, bs, H, D), lambda b, s, pref: (b, s, 0, 0)),
                pl.BlockSpec((1, bs, H, D), lambda b, s, pref: (b, s, 0, 0)),
            ],
        ),
        out_shape=[jax.ShapeDtypeStruct((B, S, H, D), jnp.float32)] * 2,
        compiler_params=pltpu.CompilerParams(
            dimension_semantics=("parallel", "parallel"),
        ),
    )(pos_adj, k, v)


def kernel(k, v, pos, max_pos, k_cache, v_cache):
    B, P, H, D = k.shape
    S = k_cache.shape[1]
    pos_i = pos.astype(jnp.int32) % S
    r = (jnp.asarray(max_pos, jnp.int32) + 1) % S
    pos_adj = (pos_i - r) % S
    ok, ov = _scatter_full(pos_adj, k, v, S)
    return ok, ov


# speedup vs baseline: 1.6515x; 1.5200x over previous
import functools

import jax
import jax.numpy as jnp
from jax.experimental import pallas as pl
from jax.experimental.pallas import tpu as pltpu


def _scatter_body(pos_ref, k_ref, v_ref, ok_ref, ov_ref, *, bs, P):
    base = pl.program_id(1) * bs
    ok_ref[...] = jnp.zeros_like(ok_ref)
    ov_ref[...] = jnp.zeros_like(ov_ref)

    def step(p, c):
        t = pos_ref[p] - base

        @pl.when((t >= 0) & (t < bs))
        def _():
            ok_ref[0, pl.ds(t, 1), :] = k_ref[0, pl.ds(p, 1), :]
            ov_ref[0, pl.ds(t, 1), :] = v_ref[0, pl.ds(p, 1), :]

        return c

    jax.lax.fori_loop(0, P, step, 0)


def _scatter_full(pos_adj, k2, v2, S, *, bs=1024):
    B, P, HD = k2.shape
    grid = (B, S // bs)
    return pl.pallas_call(
        functools.partial(_scatter_body, bs=bs, P=P),
        grid_spec=pltpu.PrefetchScalarGridSpec(
            num_scalar_prefetch=1,
            grid=grid,
            in_specs=[
                pl.BlockSpec((1, P, HD), lambda b, s, pref: (b, 0, 0)),
                pl.BlockSpec((1, P, HD), lambda b, s, pref: (b, 0, 0)),
            ],
            out_specs=[
                pl.BlockSpec((1, bs, HD), lambda b, s, pref: (b, s, 0)),
                pl.BlockSpec((1, bs, HD), lambda b, s, pref: (b, s, 0)),
            ],
        ),
        out_shape=[jax.ShapeDtypeStruct((B, S, HD), jnp.float32)] * 2,
        compiler_params=pltpu.CompilerParams(
            dimension_semantics=("parallel", "parallel"),
        ),
    )(pos_adj, k2, v2)


def kernel(k, v, pos, max_pos, k_cache, v_cache):
    B, P, H, D = k.shape
    S = k_cache.shape[1]
    HD = H * D
    pos_i = pos.astype(jnp.int32) % S
    r = (jnp.asarray(max_pos, jnp.int32) + 1) % S
    pos_adj = (pos_i - r) % S
    ok, ov = _scatter_full(pos_adj, k.reshape(B, P, HD), v.reshape(B, P, HD), S)
    return ok.reshape(B, S, H, D), ov.reshape(B, S, H, D)
